# Initial kernel scaffold; baseline (speedup 1.0000x reference)
import functools

import jax
import jax.numpy as jnp
from jax import lax
from jax.experimental import pallas as pl
from jax.experimental.pallas import tpu as pltpu
from jax.experimental.pallas import tpu_sc as plsc

L = 16
K = 128
NW = 32
N_SUB = 16
BNPOST = 1024


def _pre_body(h_ref, w_ref, b_ref, a_ref, hw_ref):
    hw = jnp.dot(h_ref[...], w_ref[...], preferred_element_type=jnp.float32)
    hw_ref[...] = hw
    a_ref[...] = hw + b_ref[...]


def _pre(h, We1, be1):
    N, D = h.shape
    BN = 400
    return pl.pallas_call(
        _pre_body,
        grid=(N // BN,),
        in_specs=[
            pl.BlockSpec((BN, D), lambda i: (i, 0)),
            pl.BlockSpec((D, D), lambda i: (0, 0)),
            pl.BlockSpec((1, D), lambda i: (0, 0)),
        ],
        out_specs=[
            pl.BlockSpec((BN, D), lambda i: (i, 0)),
            pl.BlockSpec((BN, D), lambda i: (i, 0)),
        ],
        out_shape=[
            jax.ShapeDtypeStruct((N, D), jnp.float32),
            jax.ShapeDtypeStruct((N, D), jnp.float32),
        ],
    )(h, We1, be1.reshape(1, D))


def _edge_sc(A, B, row_r, col_r, n_pad, n_blk):
    N, D = A.shape
    DW = D + L
    bpw = (n_blk + NW - 1) // NW
    rows_per_sub = n_pad // N_SUB
    mesh = plsc.VectorSubcoreMesh(core_axis_name="c", subcore_axis_name="s")

    @functools.partial(
        pl.kernel,
        out_type=jax.ShapeDtypeStruct((2, n_pad, DW), jnp.float32),
        mesh=mesh,
        scratch_types=[
            pltpu.VMEM((bpw, K), jnp.int32),
            pltpu.VMEM((bpw, K), jnp.int32),
            pltpu.VMEM((K, D), jnp.float32),
            pltpu.VMEM((K, D), jnp.float32),
            pltpu.VMEM((K, DW), jnp.float32),
            pltpu.VMEM_SHARED((n_pad, DW), jnp.float32),
            pltpu.SemaphoreType.DMA,
            pltpu.SemaphoreType.DMA,
        ],
    )
    def k(a_hbm, b_hbm, row_hbm, col_hbm, s_out, row_v, col_v, a_buf, b_buf,
          r_buf, s_acc, sem_a, sem_b):
        c = lax.axis_index("c")
        s = lax.axis_index("s")
        wid = s * 2 + c

        def zero_row(e, carry):
            for j in range(DW // L):
                r_buf[e, pl.ds(j * L, L)] = jnp.zeros((L,), jnp.float32)
            return carry
        lax.fori_loop(0, K, zero_row, 0)
        for t in range(rows_per_sub // K):
            pltpu.sync_copy(r_buf, s_acc.at[pl.ds(s * rows_per_sub + t * K, K)])

        def ones_row(e, carry):
            r_buf[e, pl.ds(D, L)] = jnp.ones((L,), jnp.float32)
            return carry
        lax.fori_loop(0, K, ones_row, 0)
        plsc.subcore_barrier()

        b0 = wid * bpw
        nb = jnp.minimum(bpw, n_blk - b0)
        pltpu.sync_copy(row_hbm.at[pl.ds(b0, bpw)], row_v)
        pltpu.sync_copy(col_hbm.at[pl.ds(b0, bpw)], col_v)

        def block(i, carry):
            da = pltpu.async_copy(a_hbm.at[row_v.at[i]], a_buf, sem_a)
            db = pltpu.async_copy(b_hbm.at[col_v.at[i]], b_buf, sem_b)
            da.wait()
            db.wait()

            def edge(e, carry2):
                for j in range(D // L):
                    av = a_buf[e, pl.ds(j * L, L)]
                    bv = b_buf[e, pl.ds(j * L, L)]
                    r_buf[e, pl.ds(j * L, L)] = jnp.maximum(av - bv, 0.0)
                return carry2
            lax.fori_loop(0, K, edge, 0)
            pltpu.sync_copy(r_buf, s_acc.at[row_v.at[i]], add=True)
            return carry
        lax.fori_loop(0, nb, block, 0)

        plsc.subcore_barrier()
        pltpu.sync_copy(s_acc.at[pl.ds(s * rows_per_sub, rows_per_sub)],
                        s_out.at[c, pl.ds(s * rows_per_sub, rows_per_sub)])

    return k(A, B, row_r, col_r)


def _post_body(h_ref, s_ref, we2_ref, wn1a_ref, wn1b_ref, wn2_ref, be2_ref,
               bn1_ref, bn2_ref, out_ref):
    ssum = s_ref[0] + s_ref[1]
    D = h_ref.shape[1]
    sm = ssum[:, :D]
    d16 = ssum[:, D:D + L]
    cvec = jnp.dot(be2_ref[...], wn1b_ref[...],
                   preferred_element_type=jnp.float32)
    m = jnp.broadcast_to(cvec / float(L), (L, D))
    aggb = (jnp.dot(jnp.dot(sm, we2_ref[...],
                            preferred_element_type=jnp.float32), wn1b_ref[...],
                    preferred_element_type=jnp.float32)
            + jnp.dot(d16, m, preferred_element_type=jnp.float32))
    pre = (jnp.dot(h_ref[...], wn1a_ref[...],
                   preferred_element_type=jnp.float32) + aggb + bn1_ref[...])
    act = jnp.maximum(pre, 0.0)
    out_ref[...] = (jnp.dot(act, wn2_ref[...],
                            preferred_element_type=jnp.float32) + bn2_ref[...])


def _post(h_pad, S, We2, Wn1a, Wn1b, Wn2, be2, bn1, bn2):
    n_pad, D = h_pad.shape
    DW = D + L
    return pl.pallas_call(
        _post_body,
        grid=(n_pad // BNPOST,),
        in_specs=[
            pl.BlockSpec((BNPOST, D), lambda i: (i, 0)),
            pl.BlockSpec((2, BNPOST, DW), lambda i: (0, i, 0)),
            pl.BlockSpec((D, D), lambda i: (0, 0)),
            pl.BlockSpec((D, D), lambda i: (0, 0)),
            pl.BlockSpec((D, D), lambda i: (0, 0)),
            pl.BlockSpec((D, D), lambda i: (0, 0)),
            pl.BlockSpec((1, D), lambda i: (0, 0)),
            pl.BlockSpec((1, D), lambda i: (0, 0)),
            pl.BlockSpec((1, D), lambda i: (0, 0)),
        ],
        out_specs=pl.BlockSpec((BNPOST, D), lambda i: (i, 0)),
        out_shape=jax.ShapeDtypeStruct((n_pad, D), jnp.float32),
    )(h_pad, S, We2, Wn1a, Wn1b, Wn2, be2.reshape(1, D), bn1.reshape(1, D),
      bn2.reshape(1, D))


def kernel(h, edge_index, We1, be1, We2, be2, Wn1, bn1, Wn2, bn2):
    N, D = h.shape
    E = edge_index.shape[1]
    n_blk = E // K
    assert n_blk * K == E and D % L == 0
    bpw = (n_blk + NW - 1) // NW
    n_blk_pad = bpw * NW
    n_pad = ((N + BNPOST - 1) // BNPOST) * BNPOST

    row = edge_index[0]
    col = edge_index[1]
    pad_e = n_blk_pad * K - E
    row_r = jnp.pad(row, (0, pad_e)).reshape(n_blk_pad, K)
    col_r = jnp.pad(col, (0, pad_e)).reshape(n_blk_pad, K)

    A, B = _pre(h, We1, be1)
    S = _edge_sc(A, B, row_r, col_r, n_pad, n_blk)

    h_pad = jnp.pad(h, ((0, n_pad - N), (0, 0)))
    out_pad = _post(h_pad, S, We2, Wn1[:D], Wn1[D:], Wn2, be2, bn1, bn2)
    return out_pad[:N]


# speedup vs baseline: 6.4817x; 6.4817x over previous
import functools

import jax
import jax.numpy as jnp
from jax import lax
from jax.experimental import pallas as pl
from jax.experimental.pallas import tpu as pltpu
from jax.experimental.pallas import tpu_sc as plsc

L = 16
K = 128
NW = 32
N_SUB = 16
BNPOST = 1024


def _pre_body(h_ref, w_ref, b_ref, a_ref, hw_ref):
    hw = jnp.dot(h_ref[...], w_ref[...], preferred_element_type=jnp.float32)
    hw_ref[...] = hw
    a_ref[...] = hw + b_ref[...]


def _pre(h, We1, be1):
    N, D = h.shape
    BN = 400
    return pl.pallas_call(
        _pre_body,
        grid=(N // BN,),
        in_specs=[
            pl.BlockSpec((BN, D), lambda i: (i, 0)),
            pl.BlockSpec((D, D), lambda i: (0, 0)),
            pl.BlockSpec((1, D), lambda i: (0, 0)),
        ],
        out_specs=[
            pl.BlockSpec((BN, D), lambda i: (i, 0)),
            pl.BlockSpec((BN, D), lambda i: (i, 0)),
        ],
        out_shape=[
            jax.ShapeDtypeStruct((N, D), jnp.float32),
            jax.ShapeDtypeStruct((N, D), jnp.float32),
        ],
    )(h, We1, be1.reshape(1, D))


def _edge_sc(A, B, row_r, col_r, n_pad, n_blk):
    N, D = A.shape
    bpw = (-(-n_blk // NW) + 7) // 8 * 8
    rows_per_sub = n_pad // N_SUB
    mesh = plsc.VectorSubcoreMesh(core_axis_name="c", subcore_axis_name="s")

    @functools.partial(
        pl.kernel,
        out_type=[
            jax.ShapeDtypeStruct((2, n_pad, D), jnp.float32),
            jax.ShapeDtypeStruct((NW, n_pad), jnp.float32),
        ],
        mesh=mesh,
        compiler_params=pltpu.CompilerParams(needs_layout_passes=False),
        scratch_types=[
            pltpu.VMEM((8, K), jnp.int32),
            pltpu.VMEM((8, K), jnp.int32),
            pltpu.VMEM((K, D), jnp.float32),
            pltpu.VMEM((K, D), jnp.float32),
            pltpu.VMEM((n_pad,), jnp.float32),
            pltpu.VMEM_SHARED((n_pad, D), jnp.float32),
            pltpu.SemaphoreType.DMA,
            pltpu.SemaphoreType.DMA,
        ],
    )
    def k(a_hbm, b_hbm, row_hbm, col_hbm, s_out, deg_out, row_v, col_v,
          a_buf, b_buf, deg_v, s_acc, sem_a, sem_b):
        c = lax.axis_index("c")
        s = lax.axis_index("s")
        wid = s * 2 + c

        def zero_row(e, carry):
            for j in range(D // L):
                a_buf[e, pl.ds(j * L, L)] = jnp.zeros((L,), jnp.float32)
            return carry
        lax.fori_loop(0, K, zero_row, 0)

        def zero_deg(t, carry):
            deg_v[pl.ds(t * L, L)] = jnp.zeros((L,), jnp.float32)
            return carry
        lax.fori_loop(0, n_pad // L, zero_deg, 0)
        sub0 = s * rows_per_sub
        for t in range(rows_per_sub // K):
            pltpu.sync_copy(a_buf, s_acc.at[pl.ds(sub0 + t * K, K)])
        plsc.subcore_barrier()

        b0 = wid * bpw
        nb = jnp.maximum(jnp.minimum(bpw, n_blk - b0), 0)
        ones_v = jnp.ones((L,), jnp.float32)

        def group(g, carry):
            pltpu.sync_copy(row_hbm.at[pl.ds(b0 + g * 8, 8)], row_v)
            pltpu.sync_copy(col_hbm.at[pl.ds(b0 + g * 8, 8)], col_v)
            ni = jnp.minimum(nb - g * 8, 8)

            def block(i, carry2):
                da = pltpu.async_copy(a_hbm.at[row_v.at[i]], a_buf, sem_a)
                db = pltpu.async_copy(b_hbm.at[col_v.at[i]], b_buf, sem_b)

                for j in range(K // L):
                    rv = row_v[i, pl.ds(j * L, L)]
                    plsc.addupdate_scatter(deg_v, [rv], ones_v)

                da.wait()
                db.wait()

                def edge(e, carry3):
                    for j in range(D // L):
                        av = a_buf[e, pl.ds(j * L, L)]
                        bv = b_buf[e, pl.ds(j * L, L)]
                        a_buf[e, pl.ds(j * L, L)] = jnp.maximum(av - bv, 0.0)
                    return carry3
                lax.fori_loop(0, K, edge, 0)

                pltpu.sync_copy(a_buf, s_acc.at[row_v.at[i]], add=True)
                return carry2
            lax.fori_loop(0, ni, block, 0)
            return carry
        lax.fori_loop(0, (nb + 7) // 8, group, 0)

        pltpu.sync_copy(deg_v, deg_out.at[wid])
        plsc.subcore_barrier()
        pltpu.sync_copy(s_acc.at[pl.ds(sub0, rows_per_sub)],
                        s_out.at[c, pl.ds(sub0, rows_per_sub)])

    return k(A, B, row_r, col_r)


def _post_body(h_ref, s_ref, d_ref, we2_ref, wn1a_ref, wn1b_ref, wn2_ref,
               be2_ref, bn1_ref, bn2_ref, out_ref):
    BN, D = h_ref.shape
    G = BN // D
    ssum = s_ref[0] + s_ref[1]
    dgrid = jnp.sum(d_ref[...], axis=0)
    rsub = lax.broadcasted_iota(jnp.int32, (BN, G), 0) // D
    rmask = (rsub == lax.broadcasted_iota(jnp.int32, (BN, G), 1)).astype(
        jnp.float32)
    u = jnp.dot(rmask, dgrid, preferred_element_type=jnp.float32)
    lsub = lax.broadcasted_iota(jnp.int32, (BN, D), 0) % D
    qmask = (lsub == lax.broadcasted_iota(jnp.int32, (BN, D), 1)).astype(
        jnp.float32)
    deg = jnp.dot(u * qmask, jnp.ones((D, D), jnp.float32),
                  preferred_element_type=jnp.float32)
    cvec = jnp.dot(be2_ref[...], wn1b_ref[...],
                   preferred_element_type=jnp.float32)
    aggb = (jnp.dot(jnp.dot(ssum, we2_ref[...],
                            preferred_element_type=jnp.float32), wn1b_ref[...],
                    preferred_element_type=jnp.float32)
            + deg * cvec)
    pre = (jnp.dot(h_ref[...], wn1a_ref[...],
                   preferred_element_type=jnp.float32) + aggb + bn1_ref[...])
    act = jnp.maximum(pre, 0.0)
    out_ref[...] = (jnp.dot(act, wn2_ref[...],
                            preferred_element_type=jnp.float32) + bn2_ref[...])


def _post(h_pad, S, deg_p, We2, Wn1a, Wn1b, Wn2, be2, bn1, bn2):
    n_pad, D = h_pad.shape
    G = BNPOST // D
    return pl.pallas_call(
        _post_body,
        grid=(n_pad // BNPOST,),
        in_specs=[
            pl.BlockSpec((BNPOST, D), lambda i: (i, 0)),
            pl.BlockSpec((2, BNPOST, D), lambda i: (0, i, 0)),
            pl.BlockSpec((NW, G, D), lambda i: (0, i, 0)),
            pl.BlockSpec((D, D), lambda i: (0, 0)),
            pl.BlockSpec((D, D), lambda i: (0, 0)),
            pl.BlockSpec((D, D), lambda i: (0, 0)),
            pl.BlockSpec((D, D), lambda i: (0, 0)),
            pl.BlockSpec((1, D), lambda i: (0, 0)),
            pl.BlockSpec((1, D), lambda i: (0, 0)),
            pl.BlockSpec((1, D), lambda i: (0, 0)),
        ],
        out_specs=pl.BlockSpec((BNPOST, D), lambda i: (i, 0)),
        out_shape=jax.ShapeDtypeStruct((h_pad.shape[0], D), jnp.float32),
    )(h_pad, S, deg_p, We2, Wn1a, Wn1b, Wn2, be2.reshape(1, D),
      bn1.reshape(1, D), bn2.reshape(1, D))


def kernel(h, edge_index, We1, be1, We2, be2, Wn1, bn1, Wn2, bn2):
    N, D = h.shape
    E = edge_index.shape[1]
    n_blk = E // K
    assert n_blk * K == E and D % L == 0
    bpw = (-(-n_blk // NW) + 7) // 8 * 8
    n_blk_pad = bpw * NW
    n_pad = ((N + BNPOST - 1) // BNPOST) * BNPOST

    row = edge_index[0]
    col = edge_index[1]
    pad_e = n_blk_pad * K - E
    row_r = jnp.pad(row, (0, pad_e)).reshape(n_blk_pad, K)
    col_r = jnp.pad(col, (0, pad_e)).reshape(n_blk_pad, K)

    A, B = _pre(h, We1, be1)
    S, deg_p = _edge_sc(A, B, row_r, col_r, n_pad, n_blk)

    h_pad = jnp.pad(h, ((0, n_pad - N), (0, 0)))
    out_pad = _post(h_pad, S, deg_p.reshape(NW, n_pad // D, D), We2,
                    Wn1[:D], Wn1[D:], Wn2, be2, bn1, bn2)
    return out_pad[:N]


# speedup vs baseline: 7.7417x; 1.1944x over previous
import functools

import jax
import jax.numpy as jnp
from jax import lax
from jax.experimental import pallas as pl
from jax.experimental.pallas import tpu as pltpu
from jax.experimental.pallas import tpu_sc as plsc

L = 16
K = 64
NW = 32
N_SUB = 16
BNPOST = 1024


def _pre_body(h_ref, w_ref, b_ref, a_ref, hw_ref):
    hw = jnp.dot(h_ref[...], w_ref[...], preferred_element_type=jnp.float32)
    hw_ref[...] = hw
    a_ref[...] = hw + b_ref[...]


def _pre(h, We1, be1):
    N, D = h.shape
    BN = 400
    return pl.pallas_call(
        _pre_body,
        grid=(N // BN,),
        in_specs=[
            pl.BlockSpec((BN, D), lambda i: (i, 0)),
            pl.BlockSpec((D, D), lambda i: (0, 0)),
            pl.BlockSpec((1, D), lambda i: (0, 0)),
        ],
        out_specs=[
            pl.BlockSpec((BN, D), lambda i: (i, 0)),
            pl.BlockSpec((BN, D), lambda i: (i, 0)),
        ],
        out_shape=[
            jax.ShapeDtypeStruct((N, D), jnp.float32),
            jax.ShapeDtypeStruct((N, D), jnp.float32),
        ],
    )(h, We1, be1.reshape(1, D))


def _edge_sc(A, B, row_r, col_r, n_pad, n_blk):
    N, D = A.shape
    bpw = (-(-n_blk // NW) + 7) // 8 * 8
    rows_per_sub = n_pad // N_SUB
    mesh = plsc.VectorSubcoreMesh(core_axis_name="c", subcore_axis_name="s")

    @functools.partial(
        pl.kernel,
        out_type=[
            jax.ShapeDtypeStruct((2, n_pad, D), jnp.float32),
            jax.ShapeDtypeStruct((NW, n_pad), jnp.float32),
        ],
        mesh=mesh,
        compiler_params=pltpu.CompilerParams(needs_layout_passes=False),
        scratch_types=[
            pltpu.VMEM((8, K), jnp.int32),
            pltpu.VMEM((8, K), jnp.int32),
            pltpu.VMEM((K, D), jnp.float32),
            pltpu.VMEM((K, D), jnp.float32),
            pltpu.VMEM((K, D), jnp.float32),
            pltpu.VMEM((K, D), jnp.float32),
            pltpu.VMEM((n_pad,), jnp.float32),
            pltpu.VMEM_SHARED((n_pad, D), jnp.float32),
            pltpu.SemaphoreType.DMA,
            pltpu.SemaphoreType.DMA,
            pltpu.SemaphoreType.DMA,
            pltpu.SemaphoreType.DMA,
        ],
    )
    def k(a_hbm, b_hbm, row_hbm, col_hbm, s_out, deg_out, row_v, col_v,
          a0_buf, b0_buf, a1_buf, b1_buf, deg_v, s_acc,
          sem_a0, sem_b0, sem_a1, sem_b1):
        c = lax.axis_index("c")
        s = lax.axis_index("s")
        wid = s * 2 + c

        def zero_row(e, carry):
            for j in range(D // L):
                a0_buf[e, pl.ds(j * L, L)] = jnp.zeros((L,), jnp.float32)
            return carry
        lax.fori_loop(0, K, zero_row, 0)

        def zero_deg(t, carry):
            deg_v[pl.ds(t * L, L)] = jnp.zeros((L,), jnp.float32)
            return carry
        lax.fori_loop(0, n_pad // L, zero_deg, 0)
        sub0 = s * rows_per_sub
        for t in range(rows_per_sub // K):
            pltpu.sync_copy(a0_buf, s_acc.at[pl.ds(sub0 + t * K, K)])
        plsc.subcore_barrier()

        b0 = wid * bpw
        nb = jnp.maximum(jnp.minimum(bpw, n_blk - b0), 0)
        ones_v = jnp.ones((L,), jnp.float32)
        sets = ((a0_buf, b0_buf, sem_a0, sem_b0),
                (a1_buf, b1_buf, sem_a1, sem_b1))

        def start(i, st):
            a_buf, b_buf, sem_a, sem_b = sets[st]
            pltpu.async_copy(a_hbm.at[row_v.at[i]], a_buf, sem_a)
            pltpu.async_copy(b_hbm.at[col_v.at[i]], b_buf, sem_b)

        def finish(i, st):
            a_buf, b_buf, sem_a, sem_b = sets[st]
            for j in range(K // L):
                rv = row_v[i, pl.ds(j * L, L)]
                plsc.addupdate_scatter(deg_v, [rv], ones_v)
            pltpu.make_async_copy(a_hbm.at[pl.ds(0, K)], a_buf, sem_a).wait()
            pltpu.make_async_copy(b_hbm.at[pl.ds(0, K)], b_buf, sem_b).wait()

            def edge(e, carry3):
                for j in range(D // L):
                    av = a_buf[e, pl.ds(j * L, L)]
                    bv = b_buf[e, pl.ds(j * L, L)]
                    a_buf[e, pl.ds(j * L, L)] = jnp.maximum(av - bv, 0.0)
                return carry3
            lax.fori_loop(0, K, edge, 0)
            pltpu.sync_copy(a_buf, s_acc.at[row_v.at[i]], add=True)

        def group(g, carry):
            pltpu.sync_copy(row_hbm.at[pl.ds(b0 + g * 8, 8)], row_v)
            pltpu.sync_copy(col_hbm.at[pl.ds(b0 + g * 8, 8)], col_v)
            ni = jnp.minimum(nb - g * 8, 8)
            start(0, 0)

            def pair(p, carry2):
                i0 = 2 * p
                start(i0 + 1, 1)
                finish(i0, 0)

                @pl.when(i0 + 2 < ni)
                def _():
                    start(i0 + 2, 0)
                finish(i0 + 1, 1)
                return carry2
            lax.fori_loop(0, ni // 2, pair, 0)
            return carry
        lax.fori_loop(0, (nb + 7) // 8, group, 0)

        pltpu.sync_copy(deg_v, deg_out.at[wid])
        plsc.subcore_barrier()
        pltpu.sync_copy(s_acc.at[pl.ds(sub0, rows_per_sub)],
                        s_out.at[c, pl.ds(sub0, rows_per_sub)])

    return k(A, B, row_r, col_r)


def _post_body(h_ref, s_ref, d_ref, we2_ref, wn1a_ref, wn1b_ref, wn2_ref,
               be2_ref, bn1_ref, bn2_ref, out_ref):
    BN, D = h_ref.shape
    G = BN // D
    ssum = s_ref[0] + s_ref[1]
    dgrid = jnp.sum(d_ref[...], axis=0)
    rsub = lax.broadcasted_iota(jnp.int32, (BN, G), 0) // D
    rmask = (rsub == lax.broadcasted_iota(jnp.int32, (BN, G), 1)).astype(
        jnp.float32)
    u = jnp.dot(rmask, dgrid, preferred_element_type=jnp.float32)
    lsub = lax.broadcasted_iota(jnp.int32, (BN, D), 0) % D
    qmask = (lsub == lax.broadcasted_iota(jnp.int32, (BN, D), 1)).astype(
        jnp.float32)
    deg = jnp.dot(u * qmask, jnp.ones((D, D), jnp.float32),
                  preferred_element_type=jnp.float32)
    cvec = jnp.dot(be2_ref[...], wn1b_ref[...],
                   preferred_element_type=jnp.float32)
    aggb = (jnp.dot(jnp.dot(ssum, we2_ref[...],
                            preferred_element_type=jnp.float32), wn1b_ref[...],
                    preferred_element_type=jnp.float32)
            + deg * cvec)
    pre = (jnp.dot(h_ref[...], wn1a_ref[...],
                   preferred_element_type=jnp.float32) + aggb + bn1_ref[...])
    act = jnp.maximum(pre, 0.0)
    out_ref[...] = (jnp.dot(act, wn2_ref[...],
                            preferred_element_type=jnp.float32) + bn2_ref[...])


def _post(h_pad, S, deg_p, We2, Wn1a, Wn1b, Wn2, be2, bn1, bn2):
    n_pad, D = h_pad.shape
    G = BNPOST // D
    return pl.pallas_call(
        _post_body,
        grid=(n_pad // BNPOST,),
        in_specs=[
            pl.BlockSpec((BNPOST, D), lambda i: (i, 0)),
            pl.BlockSpec((2, BNPOST, D), lambda i: (0, i, 0)),
            pl.BlockSpec((NW, G, D), lambda i: (0, i, 0)),
            pl.BlockSpec((D, D), lambda i: (0, 0)),
            pl.BlockSpec((D, D), lambda i: (0, 0)),
            pl.BlockSpec((D, D), lambda i: (0, 0)),
            pl.BlockSpec((D, D), lambda i: (0, 0)),
            pl.BlockSpec((1, D), lambda i: (0, 0)),
            pl.BlockSpec((1, D), lambda i: (0, 0)),
            pl.BlockSpec((1, D), lambda i: (0, 0)),
        ],
        out_specs=pl.BlockSpec((BNPOST, D), lambda i: (i, 0)),
        out_shape=jax.ShapeDtypeStruct((h_pad.shape[0], D), jnp.float32),
    )(h_pad, S, deg_p, We2, Wn1a, Wn1b, Wn2, be2.reshape(1, D),
      bn1.reshape(1, D), bn2.reshape(1, D))


def kernel(h, edge_index, We1, be1, We2, be2, Wn1, bn1, Wn2, bn2):
    N, D = h.shape
    E = edge_index.shape[1]
    n_blk = E // K
    assert n_blk * K == E and D % L == 0
    bpw = (-(-n_blk // NW) + 7) // 8 * 8
    n_blk_pad = bpw * NW
    n_pad = ((N + BNPOST - 1) // BNPOST) * BNPOST

    row = edge_index[0]
    col = edge_index[1]
    pad_e = n_blk_pad * K - E
    row_r = jnp.pad(row, (0, pad_e)).reshape(n_blk_pad, K)
    col_r = jnp.pad(col, (0, pad_e)).reshape(n_blk_pad, K)

    A, B = _pre(h, We1, be1)
    S, deg_p = _edge_sc(A, B, row_r, col_r, n_pad, n_blk)

    h_pad = jnp.pad(h, ((0, n_pad - N), (0, 0)))
    out_pad = _post(h_pad, S, deg_p.reshape(NW, n_pad // D, D), We2,
                    Wn1[:D], Wn1[D:], Wn2, be2, bn1, bn2)
    return out_pad[:N]
